# baseline (speedup 1.0000x reference)
import jax
import jax.numpy as jnp
import numpy as np
from jax.experimental import pallas as pl
from jax.experimental.pallas import tpu as pltpu
from jax.sharding import Mesh, PartitionSpec as P

try:
    _shard_map = jax.shard_map
except AttributeError:
    from jax.experimental.shard_map import shard_map as _shard_map

_K = 8
_LOSS_WEIGHT = 0.02
_BIG = 3.0e38
_IBIG = 2**30
_NS = 3
_RH = 64
_ZEPS = 1e-13


def _norm_kernel(feat_ref, out_ref):
    f = feat_ref[...]
    n2 = jnp.sum(f * f, axis=1, keepdims=True)
    n = jnp.maximum(jnp.sqrt(n2), 1e-12)
    out_ref[...] = (f / n).astype(jnp.bfloat16)


def _normalize(feat):
    n, dfeat = feat.shape
    return pl.pallas_call(
        _norm_kernel,
        grid=(n // 512,),
        in_specs=[pl.BlockSpec((512, dfeat), lambda i: (i, 0))],
        out_specs=pl.BlockSpec((512, dfeat), lambda i: (i, 0)),
        out_shape=jax.ShapeDtypeStruct((n, dfeat), jnp.bfloat16),
    )(feat)


def _knn_kernel(cb_row_ref, cb_t_ref, sq_row_ref, sq_t_ref, zcol_ref,
                featn_ref, frow_ref, acc_ref, key_ref, keep_ref):
    i = pl.program_id(0)
    R = cb_row_ref.shape[0]
    n = cb_t_ref.shape[1]

    dot = jax.lax.dot(
        cb_row_ref[...], cb_t_ref[...], preferred_element_type=jnp.float32
    )

    v0_parts, v9_parts = [], []
    for rh in range(R // _RH):
        r0 = rh * _RH
        sqr = sq_row_ref[r0 : r0 + _RH, :]
        sv = [jnp.full((_RH, 128), _BIG, jnp.float32) for _ in range(_NS)]
        for t in range(n // 128):
            c0, c1 = t * 128, (t + 1) * 128
            d2 = (sqr + sq_t_ref[:, c0:c1]) - 2.0 * dot[r0 : r0 + _RH, c0:c1]
            d2 = jnp.maximum(d2, 0.0)
            x = jnp.where(d2 == 0.0, zcol_ref[:, c0:c1], d2)
            key_ref[r0 : r0 + _RH, c0:c1] = x
            for s in range(_NS):
                lo = jnp.minimum(sv[s], x)
                if s + 1 < _NS:
                    x = jnp.maximum(sv[s], x)
                sv[s] = lo
        cand = jnp.concatenate(sv, axis=1)
        pidx = jax.lax.broadcasted_iota(jnp.int32, cand.shape, 1)
        v0 = v9 = None
        for k in range(_K + 1):
            m = jnp.min(cand, axis=1, keepdims=True)
            if k == 0:
                v0 = m
            if k == _K:
                v9 = m
            else:
                hit = cand == m
                pm = jnp.min(jnp.where(hit, pidx, _IBIG), axis=1, keepdims=True)
                cand = jnp.where(hit & (pidx == pm), _BIG, cand)
        v0_parts.append(v0)
        v9_parts.append(v9)
    v0 = jnp.concatenate(v0_parts, axis=0)
    v9 = jnp.concatenate(v9_parts, axis=0)

    key = key_ref[...]
    keep = jnp.where((key <= v9) & (key != v0), 1.0, 0.0)
    cnt = jnp.sum(keep, axis=1, keepdims=True)
    bad = jnp.sum(jnp.abs(cnt - float(_K))) != 0.0
    keep_ref[...] = keep.astype(jnp.bfloat16)

    @pl.when(bad)
    def _fallback():
        cols = jax.lax.broadcasted_iota(jnp.int32, (R, n), 1)
        kp = jnp.zeros((R, n), jnp.float32)
        for k in range(_K + 1):
            w = key_ref[...]
            m = jnp.min(w, axis=1, keepdims=True)
            hit = w == m
            jm = jnp.min(jnp.where(hit, cols, _IBIG), axis=1, keepdims=True)
            pos = hit & (cols == jm)
            if k > 0:
                kp = kp + pos.astype(jnp.float32)
            key_ref[...] = jnp.where(pos, _BIG, w)
        keep_ref[...] = kp.astype(jnp.bfloat16)

    s = jax.lax.dot(
        keep_ref[...],
        featn_ref[...],
        preferred_element_type=jnp.float32,
    )
    frow = frow_ref[...].astype(jnp.float32)
    c = jnp.sum(frow * s)

    @pl.when(i == 0)
    def _():
        acc_ref[...] = jnp.zeros_like(acc_ref)

    acc_ref[...] += c[None, None]


def _knn_rows(cb_rows, cb_t, sq_rows, sq_row_b, zcol, featn, featn_rows):
    rows = cb_rows.shape[0]
    n = cb_t.shape[1]
    dfeat = featn.shape[1]
    R = 128
    return pl.pallas_call(
        _knn_kernel,
        grid=(rows // R,),
        in_specs=[
            pl.BlockSpec((R, 3), lambda i: (i, 0)),
            pl.BlockSpec((3, n), lambda i: (0, 0)),
            pl.BlockSpec((R, 1), lambda i: (i, 0)),
            pl.BlockSpec((1, n), lambda i: (0, 0)),
            pl.BlockSpec((1, n), lambda i: (0, 0)),
            pl.BlockSpec((n, dfeat), lambda i: (0, 0)),
            pl.BlockSpec((R, dfeat), lambda i: (i, 0)),
        ],
        out_specs=pl.BlockSpec((1, 1), lambda i: (0, 0)),
        out_shape=jax.ShapeDtypeStruct((1, 1), jnp.float32),
        scratch_shapes=[
            pltpu.VMEM((R, n), jnp.float32),
            pltpu.VMEM((R, n), jnp.bfloat16),
        ],
        compiler_params=pltpu.CompilerParams(
            dimension_semantics=("arbitrary",),
        ),
    )(cb_rows, cb_t, sq_rows, sq_row_b, zcol, featn, featn_rows)


def kernel(feat_3d_list, spatial_coords_list):
    feat = feat_3d_list
    coords = spatial_coords_list
    n, dfeat = feat.shape

    cb = coords.astype(jnp.bfloat16)
    cb_t = cb.T
    sq = jnp.sum(coords * coords, axis=-1)
    sq_col = sq[:, None]
    sq_row_b = sq[None, :]
    zcol = (jnp.arange(n, dtype=jnp.float32) * _ZEPS)[None, :]

    devs = jax.devices()
    if len(devs) >= 2:
        half = n // 2
        mesh = Mesh(np.array(devs[:2]), axis_names=("x",))

        def shard_fn(cb_sh, sqc_sh, feat_rep, cbt_rep, sqr_rep, zcol_rep):
            featn = _normalize(feat_rep)
            idx = jax.lax.axis_index("x")
            featn_rows = jax.lax.dynamic_slice(
                featn, (idx * half, 0), (half, featn.shape[1])
            )
            return _knn_rows(
                cb_sh, cbt_rep, sqc_sh, sqr_rep, zcol_rep, featn, featn_rows
            )

        acc = _shard_map(
            shard_fn,
            mesh=mesh,
            check_vma=False,
            in_specs=(
                P("x", None),
                P("x", None),
                P(None, None),
                P(None, None),
                P(None, None),
                P(None, None),
            ),
            out_specs=P("x", None),
        )(cb, sq_col, feat, cb_t, sq_row_b, zcol)
        total = jnp.sum(acc)
    else:
        featn = _normalize(feat)
        acc = _knn_rows(cb, cb_t, sq_col, sq_row_b, zcol, featn, featn)
        total = acc[0, 0]

    return _LOSS_WEIGHT * (1.0 - total / (n * _K))


# speedup vs baseline: 1.4987x; 1.4987x over previous
import jax
import jax.numpy as jnp
from jax.experimental import pallas as pl
from jax.experimental.pallas import tpu as pltpu

_K = 8
_LOSS_WEIGHT = 0.02
_BIG = 3.0e38
_IBIG = 2**30
_NS = 3
_RH = 64
_ZEPS = 1e-13


def _norm_kernel(feat_ref, out_ref):
    f = feat_ref[...]
    n2 = jnp.sum(f * f, axis=1, keepdims=True)
    n = jnp.maximum(jnp.sqrt(n2), 1e-12)
    out_ref[...] = (f / n).astype(jnp.bfloat16)


def _normalize(feat):
    n, dfeat = feat.shape
    return pl.pallas_call(
        _norm_kernel,
        grid=(n // 512,),
        in_specs=[pl.BlockSpec((512, dfeat), lambda i: (i, 0))],
        out_specs=pl.BlockSpec((512, dfeat), lambda i: (i, 0)),
        out_shape=jax.ShapeDtypeStruct((n, dfeat), jnp.bfloat16),
    )(feat)


def _knn_kernel(cb_row_ref, cb_t_ref, sq_row_ref, sq_t_ref, zcol_ref,
                featn_ref, frow_ref, acc_ref, key_ref, keep_ref):
    i = pl.program_id(0)
    R = cb_row_ref.shape[0]
    n = cb_t_ref.shape[1]

    dot = jax.lax.dot(
        cb_row_ref[...], cb_t_ref[...], preferred_element_type=jnp.float32
    )

    v0_parts, v9_parts = [], []
    for rh in range(R // _RH):
        r0 = rh * _RH
        sqr = sq_row_ref[r0 : r0 + _RH, :]
        sv = [jnp.full((_RH, 128), _BIG, jnp.float32) for _ in range(_NS)]
        for t in range(n // 128):
            c0, c1 = t * 128, (t + 1) * 128
            d2 = (sqr + sq_t_ref[:, c0:c1]) - 2.0 * dot[r0 : r0 + _RH, c0:c1]
            d2 = jnp.maximum(d2, 0.0)
            x = jnp.where(d2 == 0.0, zcol_ref[:, c0:c1], d2)
            key_ref[r0 : r0 + _RH, c0:c1] = x
            for s in range(_NS):
                lo = jnp.minimum(sv[s], x)
                if s + 1 < _NS:
                    x = jnp.maximum(sv[s], x)
                sv[s] = lo
        cand = jnp.concatenate(sv, axis=1)
        pidx = jax.lax.broadcasted_iota(jnp.int32, cand.shape, 1)
        v0 = v9 = None
        for k in range(_K + 1):
            m = jnp.min(cand, axis=1, keepdims=True)
            if k == 0:
                v0 = m
            if k == _K:
                v9 = m
            else:
                hit = cand == m
                pm = jnp.min(jnp.where(hit, pidx, _IBIG), axis=1, keepdims=True)
                cand = jnp.where(hit & (pidx == pm), _BIG, cand)
        v0_parts.append(v0)
        v9_parts.append(v9)
    v0 = jnp.concatenate(v0_parts, axis=0)
    v9 = jnp.concatenate(v9_parts, axis=0)

    key = key_ref[...]
    keep = jnp.where((key <= v9) & (key != v0), 1.0, 0.0)
    cnt = jnp.sum(keep, axis=1, keepdims=True)
    bad = jnp.sum(jnp.abs(cnt - float(_K))) != 0.0
    keep_ref[...] = keep.astype(jnp.bfloat16)

    @pl.when(bad)
    def _fallback():
        cols = jax.lax.broadcasted_iota(jnp.int32, (R, n), 1)
        kp = jnp.zeros((R, n), jnp.float32)
        for k in range(_K + 1):
            w = key_ref[...]
            m = jnp.min(w, axis=1, keepdims=True)
            hit = w == m
            jm = jnp.min(jnp.where(hit, cols, _IBIG), axis=1, keepdims=True)
            pos = hit & (cols == jm)
            if k > 0:
                kp = kp + pos.astype(jnp.float32)
            key_ref[...] = jnp.where(pos, _BIG, w)
        keep_ref[...] = kp.astype(jnp.bfloat16)

    s = jax.lax.dot(
        keep_ref[...],
        featn_ref[...],
        preferred_element_type=jnp.float32,
    )
    frow = frow_ref[...].astype(jnp.float32)
    c = jnp.sum(frow * s)

    @pl.when(i == 0)
    def _():
        acc_ref[...] = jnp.zeros_like(acc_ref)

    acc_ref[...] += c[None, None]


def _knn_rows(cb_rows, cb_t, sq_rows, sq_row_b, zcol, featn, featn_rows):
    rows = cb_rows.shape[0]
    n = cb_t.shape[1]
    dfeat = featn.shape[1]
    R = 128
    return pl.pallas_call(
        _knn_kernel,
        grid=(rows // R,),
        in_specs=[
            pl.BlockSpec((R, 3), lambda i: (i, 0)),
            pl.BlockSpec((3, n), lambda i: (0, 0)),
            pl.BlockSpec((R, 1), lambda i: (i, 0)),
            pl.BlockSpec((1, n), lambda i: (0, 0)),
            pl.BlockSpec((1, n), lambda i: (0, 0)),
            pl.BlockSpec((n, dfeat), lambda i: (0, 0)),
            pl.BlockSpec((R, dfeat), lambda i: (i, 0)),
        ],
        out_specs=pl.BlockSpec((1, 1), lambda i: (0, 0)),
        out_shape=jax.ShapeDtypeStruct((1, 1), jnp.float32),
        scratch_shapes=[
            pltpu.VMEM((R, n), jnp.float32),
            pltpu.VMEM((R, n), jnp.bfloat16),
        ],
        compiler_params=pltpu.CompilerParams(
            dimension_semantics=("arbitrary",),
        ),
    )(cb_rows, cb_t, sq_rows, sq_row_b, zcol, featn, featn_rows)


def kernel(feat_3d_list, spatial_coords_list):
    feat = feat_3d_list
    coords = spatial_coords_list
    n, dfeat = feat.shape

    cb = coords.astype(jnp.bfloat16)
    cb_t = cb.T
    sq = jnp.sum(coords * coords, axis=-1)
    sq_col = sq[:, None]
    sq_row_b = sq[None, :]
    zcol = (jnp.arange(n, dtype=jnp.float32) * _ZEPS)[None, :]

    featn = _normalize(feat)
    acc = _knn_rows(cb, cb_t, sq_col, sq_row_b, zcol, featn, featn)
    total = acc[0, 0]

    return _LOSS_WEIGHT * (1.0 - total / (n * _K))


# speedup vs baseline: 1.5570x; 1.0389x over previous
import jax
import jax.numpy as jnp
from jax.experimental import pallas as pl
from jax.experimental.pallas import tpu as pltpu

_K = 8
_LOSS_WEIGHT = 0.02
_BIG = 3.0e38
_IBIG = 2**30
_NS = 3
_RH = 64
_ZEPS = 1e-13


def _norm_kernel(feat_ref, out_ref):
    f = feat_ref[...]
    n2 = jnp.sum(f * f, axis=1, keepdims=True)
    n = jnp.maximum(jnp.sqrt(n2), 1e-12)
    out_ref[...] = (f / n).astype(jnp.bfloat16)


def _normalize(feat):
    n, dfeat = feat.shape
    return pl.pallas_call(
        _norm_kernel,
        grid=(n // 512,),
        in_specs=[pl.BlockSpec((512, dfeat), lambda i: (i, 0))],
        out_specs=pl.BlockSpec((512, dfeat), lambda i: (i, 0)),
        out_shape=jax.ShapeDtypeStruct((n, dfeat), jnp.bfloat16),
    )(feat)


def _knn_kernel(cb_row_ref, cb_t_ref, sq_row_ref, sq_t_ref, zcol_ref,
                featn_ref, frow_ref, acc_ref, key_ref, keep_ref):
    i = pl.program_id(0)
    R = cb_row_ref.shape[0]
    n = cb_t_ref.shape[1]

    dot = jax.lax.dot(
        cb_row_ref[...], cb_t_ref[...], preferred_element_type=jnp.float32
    )

    v0_parts, v9_parts = [], []
    for rh in range(R // _RH):
        r0 = rh * _RH
        sqr = sq_row_ref[r0 : r0 + _RH, :]
        sv = [jnp.full((_RH, 128), _BIG, jnp.float32) for _ in range(_NS)]
        for t in range(n // 128):
            c0, c1 = t * 128, (t + 1) * 128
            d2 = (sqr + sq_t_ref[:, c0:c1]) - 2.0 * dot[r0 : r0 + _RH, c0:c1]
            x = jnp.maximum(d2, zcol_ref[:, c0:c1])
            key_ref[r0 : r0 + _RH, c0:c1] = x
            for s in range(_NS):
                lo = jnp.minimum(sv[s], x)
                if s + 1 < _NS:
                    x = jnp.maximum(sv[s], x)
                sv[s] = lo
        cand = jnp.concatenate(sv, axis=1)
        pidx = jax.lax.broadcasted_iota(jnp.int32, cand.shape, 1)
        v0 = v9 = None
        for k in range(_K + 1):
            m = jnp.min(cand, axis=1, keepdims=True)
            if k == 0:
                v0 = m
            if k == _K:
                v9 = m
            else:
                hit = cand == m
                pm = jnp.min(jnp.where(hit, pidx, _IBIG), axis=1, keepdims=True)
                cand = jnp.where(hit & (pidx == pm), _BIG, cand)
        v0_parts.append(v0)
        v9_parts.append(v9)
    v0 = jnp.concatenate(v0_parts, axis=0)
    v9 = jnp.concatenate(v9_parts, axis=0)

    key = key_ref[...]
    keep = jnp.where((key <= v9) & (key != v0), 1.0, 0.0)
    cnt = jnp.sum(keep, axis=1, keepdims=True)
    bad = jnp.sum(jnp.abs(cnt - float(_K))) != 0.0
    keep_ref[...] = keep.astype(jnp.bfloat16)

    @pl.when(bad)
    def _fallback():
        cols = jax.lax.broadcasted_iota(jnp.int32, (R, n), 1)
        kp = jnp.zeros((R, n), jnp.float32)
        for k in range(_K + 1):
            w = key_ref[...]
            m = jnp.min(w, axis=1, keepdims=True)
            hit = w == m
            jm = jnp.min(jnp.where(hit, cols, _IBIG), axis=1, keepdims=True)
            pos = hit & (cols == jm)
            if k > 0:
                kp = kp + pos.astype(jnp.float32)
            key_ref[...] = jnp.where(pos, _BIG, w)
        keep_ref[...] = kp.astype(jnp.bfloat16)

    s = jax.lax.dot(
        keep_ref[...],
        featn_ref[...],
        preferred_element_type=jnp.float32,
    )
    frow = frow_ref[...].astype(jnp.float32)
    c = jnp.sum(frow * s)

    @pl.when(i == 0)
    def _():
        acc_ref[...] = jnp.zeros_like(acc_ref)

    acc_ref[...] += c[None, None]


def _knn_rows(cb_rows, cb_t, sq_rows, sq_row_b, zcol, featn, featn_rows):
    rows = cb_rows.shape[0]
    n = cb_t.shape[1]
    dfeat = featn.shape[1]
    R = 128
    return pl.pallas_call(
        _knn_kernel,
        grid=(rows // R,),
        in_specs=[
            pl.BlockSpec((R, 3), lambda i: (i, 0)),
            pl.BlockSpec((3, n), lambda i: (0, 0)),
            pl.BlockSpec((R, 1), lambda i: (i, 0)),
            pl.BlockSpec((1, n), lambda i: (0, 0)),
            pl.BlockSpec((1, n), lambda i: (0, 0)),
            pl.BlockSpec((n, dfeat), lambda i: (0, 0)),
            pl.BlockSpec((R, dfeat), lambda i: (i, 0)),
        ],
        out_specs=pl.BlockSpec((1, 1), lambda i: (0, 0)),
        out_shape=jax.ShapeDtypeStruct((1, 1), jnp.float32),
        scratch_shapes=[
            pltpu.VMEM((R, n), jnp.float32),
            pltpu.VMEM((R, n), jnp.bfloat16),
        ],
        compiler_params=pltpu.CompilerParams(
            dimension_semantics=("arbitrary",),
        ),
    )(cb_rows, cb_t, sq_rows, sq_row_b, zcol, featn, featn_rows)


def kernel(feat_3d_list, spatial_coords_list):
    feat = feat_3d_list
    coords = spatial_coords_list
    n, dfeat = feat.shape

    cb = coords.astype(jnp.bfloat16)
    cb_t = cb.T
    sq = jnp.sum(coords * coords, axis=-1)
    sq_col = sq[:, None]
    sq_row_b = sq[None, :]
    zcol = (jnp.arange(n, dtype=jnp.float32) * _ZEPS)[None, :]

    featn = _normalize(feat)
    acc = _knn_rows(cb, cb_t, sq_col, sq_row_b, zcol, featn, featn)
    total = acc[0, 0]

    return _LOSS_WEIGHT * (1.0 - total / (n * _K))


# speedup vs baseline: 1.6503x; 1.0599x over previous
import jax
import jax.numpy as jnp
from jax.experimental import pallas as pl
from jax.experimental.pallas import tpu as pltpu

_K = 8
_LOSS_WEIGHT = 0.02
_BIG = 3.0e38
_IBIG = 2**30
_NS = 3
_RH = 64
_ZEPS = 1e-13


def _norm_kernel(feat_ref, out_ref):
    f = feat_ref[...]
    n2 = jnp.sum(f * f, axis=1, keepdims=True)
    n = jnp.maximum(jnp.sqrt(n2), 1e-12)
    out_ref[...] = (f / n).astype(jnp.bfloat16)


def _normalize(feat):
    n, dfeat = feat.shape
    return pl.pallas_call(
        _norm_kernel,
        grid=(n // 512,),
        in_specs=[pl.BlockSpec((512, dfeat), lambda i: (i, 0))],
        out_specs=pl.BlockSpec((512, dfeat), lambda i: (i, 0)),
        out_shape=jax.ShapeDtypeStruct((n, dfeat), jnp.bfloat16),
    )(feat)


def _knn_kernel(cb_row_ref, cb_t_ref, sq_row_ref, sq_t_ref, zcol_ref,
                featn_ref, frow_ref, acc_ref, key_ref, keep_ref):
    i = pl.program_id(0)
    R = cb_row_ref.shape[0]
    n = cb_t_ref.shape[1]

    v0_parts, v9_parts = [], []
    for rh in range(R // _RH):
        r0 = rh * _RH
        cbr = cb_row_ref[r0 : r0 + _RH, :]
        sqr = sq_row_ref[r0 : r0 + _RH, :]
        sv = [jnp.full((_RH, 128), _BIG, jnp.float32) for _ in range(_NS)]
        for t in range(n // 128):
            c0, c1 = t * 128, (t + 1) * 128
            dot = jax.lax.dot(
                cbr, cb_t_ref[:, c0:c1], preferred_element_type=jnp.float32
            )
            d2 = (sqr + sq_t_ref[:, c0:c1]) - 2.0 * dot
            x = jnp.maximum(d2, zcol_ref[:, c0:c1])
            key_ref[r0 : r0 + _RH, c0:c1] = x
            for s in range(_NS):
                lo = jnp.minimum(sv[s], x)
                if s + 1 < _NS:
                    x = jnp.maximum(sv[s], x)
                sv[s] = lo
        cand = jnp.concatenate(sv, axis=1)
        pidx = jax.lax.broadcasted_iota(jnp.int32, cand.shape, 1)
        v0 = v9 = None
        for k in range(_K + 1):
            m = jnp.min(cand, axis=1, keepdims=True)
            if k == 0:
                v0 = m
            if k == _K:
                v9 = m
            else:
                hit = cand == m
                pm = jnp.min(jnp.where(hit, pidx, _IBIG), axis=1, keepdims=True)
                cand = jnp.where(hit & (pidx == pm), _BIG, cand)
        v0_parts.append(v0)
        v9_parts.append(v9)
    v0 = jnp.concatenate(v0_parts, axis=0)
    v9 = jnp.concatenate(v9_parts, axis=0)

    key = key_ref[...]
    keep = jnp.where((key <= v9) & (key != v0), 1.0, 0.0)
    cnt = jnp.sum(keep, axis=1, keepdims=True)
    bad = jnp.sum(jnp.abs(cnt - float(_K))) != 0.0
    keep_ref[...] = keep.astype(jnp.bfloat16)

    @pl.when(bad)
    def _fallback():
        cols = jax.lax.broadcasted_iota(jnp.int32, (R, n), 1)
        kp = jnp.zeros((R, n), jnp.float32)
        for k in range(_K + 1):
            w = key_ref[...]
            m = jnp.min(w, axis=1, keepdims=True)
            hit = w == m
            jm = jnp.min(jnp.where(hit, cols, _IBIG), axis=1, keepdims=True)
            pos = hit & (cols == jm)
            if k > 0:
                kp = kp + pos.astype(jnp.float32)
            key_ref[...] = jnp.where(pos, _BIG, w)
        keep_ref[...] = kp.astype(jnp.bfloat16)

    s = jax.lax.dot(
        keep_ref[...],
        featn_ref[...],
        preferred_element_type=jnp.float32,
    )
    frow = frow_ref[...].astype(jnp.float32)
    c = jnp.sum(frow * s)

    @pl.when(i == 0)
    def _():
        acc_ref[...] = jnp.zeros_like(acc_ref)

    acc_ref[...] += c[None, None]


def _knn_rows(cb_rows, cb_t, sq_rows, sq_row_b, zcol, featn, featn_rows):
    rows = cb_rows.shape[0]
    n = cb_t.shape[1]
    dfeat = featn.shape[1]
    R = 128
    return pl.pallas_call(
        _knn_kernel,
        grid=(rows // R,),
        in_specs=[
            pl.BlockSpec((R, 3), lambda i: (i, 0)),
            pl.BlockSpec((3, n), lambda i: (0, 0)),
            pl.BlockSpec((R, 1), lambda i: (i, 0)),
            pl.BlockSpec((1, n), lambda i: (0, 0)),
            pl.BlockSpec((1, n), lambda i: (0, 0)),
            pl.BlockSpec((n, dfeat), lambda i: (0, 0)),
            pl.BlockSpec((R, dfeat), lambda i: (i, 0)),
        ],
        out_specs=pl.BlockSpec((1, 1), lambda i: (0, 0)),
        out_shape=jax.ShapeDtypeStruct((1, 1), jnp.float32),
        scratch_shapes=[
            pltpu.VMEM((R, n), jnp.float32),
            pltpu.VMEM((R, n), jnp.bfloat16),
        ],
        compiler_params=pltpu.CompilerParams(
            dimension_semantics=("arbitrary",),
        ),
    )(cb_rows, cb_t, sq_rows, sq_row_b, zcol, featn, featn_rows)


def kernel(feat_3d_list, spatial_coords_list):
    feat = feat_3d_list
    coords = spatial_coords_list
    n, dfeat = feat.shape

    cb = coords.astype(jnp.bfloat16)
    cb_t = cb.T
    sq = jnp.sum(coords * coords, axis=-1)
    sq_col = sq[:, None]
    sq_row_b = sq[None, :]
    zcol = (jnp.arange(n, dtype=jnp.float32) * _ZEPS)[None, :]

    featn = _normalize(feat)
    acc = _knn_rows(cb, cb_t, sq_col, sq_row_b, zcol, featn, featn)
    total = acc[0, 0]

    return _LOSS_WEIGHT * (1.0 - total / (n * _K))


# speedup vs baseline: 1.6510x; 1.0004x over previous
import jax
import jax.numpy as jnp
from jax.experimental import pallas as pl
from jax.experimental.pallas import tpu as pltpu

_K = 8
_LOSS_WEIGHT = 0.02
_BIG = 3.0e38
_IBIG = 2**30
_NS = 3
_RH = 64
_ZEPS = 1e-13


def _norm_kernel(feat_ref, out_ref):
    f = feat_ref[...]
    n2 = jnp.sum(f * f, axis=1, keepdims=True)
    n = jnp.maximum(jnp.sqrt(n2), 1e-12)
    out_ref[...] = (f / n).astype(jnp.bfloat16)


def _normalize(feat):
    n, dfeat = feat.shape
    return pl.pallas_call(
        _norm_kernel,
        grid=(n // 512,),
        in_specs=[pl.BlockSpec((512, dfeat), lambda i: (i, 0))],
        out_specs=pl.BlockSpec((512, dfeat), lambda i: (i, 0)),
        out_shape=jax.ShapeDtypeStruct((n, dfeat), jnp.bfloat16),
    )(feat)


def _knn_kernel(cb_row_ref, cb_t_ref, sq_row_ref, sq_t_ref, zcol_ref,
                featn_ref, frow_ref, acc_ref, key_ref, keep_ref):
    i = pl.program_id(0)
    R = cb_row_ref.shape[0]
    n = cb_t_ref.shape[1]

    v0_parts, v9_parts = [], []
    for rh in range(R // _RH):
        r0 = rh * _RH
        cbr = cb_row_ref[r0 : r0 + _RH, :]
        sqr = sq_row_ref[r0 : r0 + _RH, :]
        sv = [jnp.full((_RH, 128), _BIG, jnp.float32) for _ in range(_NS)]
        for t in range(n // 128):
            c0, c1 = t * 128, (t + 1) * 128
            dot2 = jax.lax.dot(
                cbr, cb_t_ref[:, c0:c1], preferred_element_type=jnp.float32
            )
            d2 = (sqr + sq_t_ref[:, c0:c1]) - dot2
            x = jnp.maximum(d2, zcol_ref[:, c0:c1])
            key_ref[r0 : r0 + _RH, c0:c1] = x
            for s in range(_NS):
                lo = jnp.minimum(sv[s], x)
                if s + 1 < _NS:
                    x = jnp.maximum(sv[s], x)
                sv[s] = lo
        cand = jnp.concatenate(sv, axis=1)
        pidx = jax.lax.broadcasted_iota(jnp.int32, cand.shape, 1)
        v0 = v9 = None
        for k in range(_K + 1):
            m = jnp.min(cand, axis=1, keepdims=True)
            if k == 0:
                v0 = m
            if k == _K:
                v9 = m
            else:
                hit = cand == m
                pm = jnp.min(jnp.where(hit, pidx, _IBIG), axis=1, keepdims=True)
                cand = jnp.where(hit & (pidx == pm), _BIG, cand)
        v0_parts.append(v0)
        v9_parts.append(v9)
    v0 = jnp.concatenate(v0_parts, axis=0)
    v9 = jnp.concatenate(v9_parts, axis=0)

    key = key_ref[...]
    keep = jnp.where((key <= v9) & (key != v0), 1.0, 0.0)
    cnt = jnp.sum(keep, axis=1, keepdims=True)
    bad = jnp.sum(jnp.abs(cnt - float(_K))) != 0.0
    keep_ref[...] = keep.astype(jnp.bfloat16)

    @pl.when(bad)
    def _fallback():
        cols = jax.lax.broadcasted_iota(jnp.int32, (R, n), 1)
        kp = jnp.zeros((R, n), jnp.float32)
        for k in range(_K + 1):
            w = key_ref[...]
            m = jnp.min(w, axis=1, keepdims=True)
            hit = w == m
            jm = jnp.min(jnp.where(hit, cols, _IBIG), axis=1, keepdims=True)
            pos = hit & (cols == jm)
            if k > 0:
                kp = kp + pos.astype(jnp.float32)
            key_ref[...] = jnp.where(pos, _BIG, w)
        keep_ref[...] = kp.astype(jnp.bfloat16)

    s = jax.lax.dot(
        keep_ref[...],
        featn_ref[...],
        preferred_element_type=jnp.float32,
    )
    frow = frow_ref[...].astype(jnp.float32)
    c = jnp.sum(frow * s)

    @pl.when(i == 0)
    def _():
        acc_ref[...] = jnp.zeros_like(acc_ref)

    acc_ref[...] += c[None, None]


def _knn_rows(cb_rows, cb_t, sq_rows, sq_row_b, zcol, featn, featn_rows):
    rows = cb_rows.shape[0]
    n = cb_t.shape[1]
    dfeat = featn.shape[1]
    R = 128
    return pl.pallas_call(
        _knn_kernel,
        grid=(rows // R,),
        in_specs=[
            pl.BlockSpec((R, 3), lambda i: (i, 0)),
            pl.BlockSpec((3, n), lambda i: (0, 0)),
            pl.BlockSpec((R, 1), lambda i: (i, 0)),
            pl.BlockSpec((1, n), lambda i: (0, 0)),
            pl.BlockSpec((1, n), lambda i: (0, 0)),
            pl.BlockSpec((n, dfeat), lambda i: (0, 0)),
            pl.BlockSpec((R, dfeat), lambda i: (i, 0)),
        ],
        out_specs=pl.BlockSpec((1, 1), lambda i: (0, 0)),
        out_shape=jax.ShapeDtypeStruct((1, 1), jnp.float32),
        scratch_shapes=[
            pltpu.VMEM((R, n), jnp.float32),
            pltpu.VMEM((R, n), jnp.bfloat16),
        ],
        compiler_params=pltpu.CompilerParams(
            dimension_semantics=("arbitrary",),
        ),
    )(cb_rows, cb_t, sq_rows, sq_row_b, zcol, featn, featn_rows)


def kernel(feat_3d_list, spatial_coords_list):
    feat = feat_3d_list
    coords = spatial_coords_list
    n, dfeat = feat.shape

    cb = coords.astype(jnp.bfloat16)
    cb_t = cb.T
    cb2 = cb * jnp.bfloat16(2.0)
    sq = jnp.sum(coords * coords, axis=-1)
    sq_col = sq[:, None]
    sq_row_b = sq[None, :]
    zcol = (jnp.arange(n, dtype=jnp.float32) * _ZEPS)[None, :]

    featn = _normalize(feat)
    acc = _knn_rows(cb2, cb_t, sq_col, sq_row_b, zcol, featn, featn)
    total = acc[0, 0]

    return _LOSS_WEIGHT * (1.0 - total / (n * _K))
